# baseline (speedup 1.0000x reference)
import functools

import jax
import jax.numpy as jnp
from jax import lax
from jax.experimental import pallas as pl
from jax.experimental.pallas import tpu as pltpu
from jax.experimental.pallas import tpu_sc as plsc

_VEC = 16
_CS = 8


def kernel(x, pe):
    B, S, D = x.shape
    info = plsc.get_sparse_core_info()
    nw = info.num_cores * info.num_subcores
    s_per_w = S // nw
    n_chunks = s_per_w // _CS
    chunk_words = _CS * D

    xf = x.reshape(B, S * D)
    pef = pe.reshape(S * D)
    mesh = plsc.VectorSubcoreMesh(core_axis_name="c", subcore_axis_name="s")

    @functools.partial(
        pl.kernel,
        mesh=mesh,
        out_type=jax.ShapeDtypeStruct((B, S * D), jnp.float32),
        scratch_types=[
            pltpu.VMEM((chunk_words,), jnp.float32),
            pltpu.VMEM((B, chunk_words), jnp.float32),
            pltpu.SemaphoreType.DMA,
        ],
    )
    def sc_add(x_hbm, pe_hbm, out_hbm, pe_v, x_v, sem):
        wid = lax.axis_index("s") * info.num_cores + lax.axis_index("c")
        s_base = wid * s_per_w

        def chunk_body(c, _):
            off = (s_base + c * _CS) * D
            cp_pe = pltpu.async_copy(pe_hbm.at[pl.ds(off, chunk_words)], pe_v, sem)
            cps = [
                pltpu.async_copy(x_hbm.at[b, pl.ds(off, chunk_words)], x_v.at[b], sem)
                for b in range(B)
            ]
            cp_pe.wait()
            for cp in cps:
                cp.wait()

            def vec_body(j, _):
                sl = pl.ds(j * _VEC, _VEC)
                pv = pe_v[sl]
                for b in range(B):
                    x_v[b, sl] += pv
                return 0

            lax.fori_loop(0, chunk_words // _VEC, vec_body, 0)
            for b in range(B):
                pltpu.sync_copy(x_v.at[b], out_hbm.at[b, pl.ds(off, chunk_words)])
            return 0

        lax.fori_loop(0, n_chunks, chunk_body, 0)

    out = sc_add(xf, pef)
    return out.reshape(B, S, D)
